# baseline (speedup 1.0000x reference)
import jax
import jax.numpy as jnp
from jax.experimental import pallas as pl
from jax.experimental.pallas import tpu as pltpu

_EPS = 1e-5
_IMGS = 16
_GRP = 4
_OFF = 16


def _fold3x3(w, W):
    shift = jnp.stack([jnp.eye(W, W, k=1 - dx, dtype=w.dtype) for dx in range(3)])
    band = jnp.einsum("dij,ydab->yiajb", shift, w)
    return band.reshape(3, W * w.shape[2], W * w.shape[3])


def _fold1x1(w, W):
    return jnp.kron(jnp.eye(W, dtype=w.dtype), w)


def _fold1x1_s2(w, W):
    pick = jnp.eye(W, dtype=w.dtype)[:, ::2]
    return jnp.einsum("ij,ab->iajb", pick, w).reshape(W * w.shape[0], (W // 2) * w.shape[1])


def _row(v, W):
    return jnp.tile(v.astype(jnp.float32), W)[None, :]


def _dot(a, b):
    return jnp.dot(a, b, preferred_element_type=jnp.float32)


def _scatter(pad_ref, imgs, H):
    seg = H + _OFF
    zero = jnp.zeros((1, pad_ref.shape[1]), pad_ref.dtype)
    for i, img in enumerate(imgs):
        base = i * seg
        pad_ref[base:base + 1, :] = zero
        pad_ref[base + _OFF - 1:base + _OFF, :] = zero
        pad_ref[base + _OFF:base + _OFF + H, :] = img
    pad_ref[len(imgs) * seg:len(imgs) * seg + 1, :] = zero


def _band_conv(pad_ref, w_ref, M):
    acc = _dot(pad_ref[_OFF - 1:_OFF - 1 + M, :], w_ref[0])
    acc += _dot(pad_ref[_OFF:_OFF + M, :], w_ref[1])
    acc += _dot(pad_ref[_OFF + 1:_OFF + 1 + M, :], w_ref[2])
    return acc


def _phase1(x_ref, w1_ref, b1_ref, wb1_ref, bb1_ref, t_ref, st_ref,
            pa_ref, pb_ref):
    B, H = x_ref.shape[0], x_ref.shape[1]
    seg = H + _OFF
    P = min(_GRP, B)
    M = (P - 1) * seg + H + 2

    s1 = jnp.zeros((1, t_ref.shape[2]), jnp.float32)
    s2 = jnp.zeros((1, t_ref.shape[2]), jnp.float32)
    for g in range(0, B, P):
        _scatter(pa_ref, [x_ref[g + i].astype(pa_ref.dtype) for i in range(P)], H)
        x1 = jnp.maximum(_band_conv(pa_ref, w1_ref, M) + b1_ref[...], 0.0)

        _scatter(pb_ref, [x1[i * seg:i * seg + H].astype(pb_ref.dtype)
                          for i in range(P)], H)
        t = _band_conv(pb_ref, wb1_ref, M) + bb1_ref[...]

        for i in range(P):
            ti = t[i * seg:i * seg + H]
            t_ref[g + i] = ti.astype(t_ref.dtype)
            s1 += jnp.sum(ti, axis=0, keepdims=True)
            s2 += jnp.sum(ti * ti, axis=0, keepdims=True)
    st_ref[0:1, :] = s1
    st_ref[1:2, :] = s2


def _phase2(t_ref, x_ref, sc_ref, sh_ref, wb2_ref, bb2_ref, ws_ref, bs_ref,
            w2_ref, b2_ref, sel_ref, wd_ref, bd_ref, out_ref, dn_ref, pa_ref):
    B, H = t_ref.shape[0], t_ref.shape[1]
    seg = H + _OFF
    P = min(_GRP, B)
    M = (P - 1) * seg + H + 2
    Hh = H // 2
    bf16 = jnp.bfloat16

    for g in range(0, B, P):
        tb = [jnp.maximum(t_ref[g + i].astype(jnp.float32) * sc_ref[...]
                          + sh_ref[...], 0.0).astype(bf16) for i in range(P)]
        _scatter(pa_ref, tb, H)
        acc = _band_conv(pa_ref, wb2_ref, M) + bb2_ref[...]

        xs = _dot(x_ref[g:g + P].reshape(P * H, x_ref.shape[2]).astype(bf16),
                  ws_ref[...]) + bs_ref[...]

        x3 = [(acc[i * seg:i * seg + H] + xs[i * H:(i + 1) * H]).astype(bf16)
              for i in range(P)]
        _scatter(pa_ref, x3, H)
        out = jnp.maximum(_band_conv(pa_ref, w2_ref, M) + b2_ref[...], 0.0)

        dec = []
        for i in range(P):
            oi = out[i * seg:i * seg + H]
            out_ref[g + i] = oi
            dec.append(_dot(sel_ref[...], oi.astype(bf16)).astype(bf16))
        dn = _dot(jnp.concatenate(dec, axis=0), wd_ref[...]) + bd_ref[...]
        for i in range(P):
            dn_ref[g + i] = dn[i * Hh:(i + 1) * Hh]


def kernel(x_nhwc, w1, b1, wb1, bb1, gamma, beta, wb2, bb2, ws, bs, w2, b2, wd, bd):
    N, H, W, Cin = x_nhwc.shape
    Cout = w1.shape[-1]
    f32, bf16 = jnp.float32, jnp.bfloat16
    Wci, Wco, Wcd = W * Cin, W * Cout, (W // 2) * Cout

    B = _IMGS
    while N % B:
        B //= 2
    G = N // B
    rows = min(_GRP, B) * (H + _OFF) + 16

    x2d = x_nhwc.reshape(N, H, Wci)

    w1b = _fold3x3(w1, W).astype(bf16)
    wb1b = _fold3x3(wb1, W).astype(bf16)
    wb2b = _fold3x3(wb2, W).astype(bf16)
    w2b = _fold3x3(w2, W).astype(bf16)
    wsb = _fold1x1(ws, W).astype(bf16)
    wdb = _fold1x1_s2(wd, W).astype(bf16)
    selb = jnp.eye(H, dtype=bf16)[::2, :]
    b1r, bb1r = _row(b1, W), _row(bb1, W)
    bb2r, bsr = _row(bb2, W), _row(bs, W)
    b2r, bdr = _row(b2, W), _row(bd, W // 2)

    par = pltpu.CompilerParams(dimension_semantics=("parallel",))
    rep2 = lambda g: (0, 0)
    rep3 = lambda g: (0, 0, 0)
    blk = lambda g: (g, 0, 0)

    t2d, stats = pl.pallas_call(
        _phase1,
        out_shape=(jax.ShapeDtypeStruct((N, H, Wco), bf16),
                   jax.ShapeDtypeStruct((G, 2, Wco), f32)),
        grid=(G,),
        in_specs=[
            pl.BlockSpec((B, H, Wci), blk),
            pl.BlockSpec((3, Wci, Wco), rep3),
            pl.BlockSpec((1, Wco), rep2),
            pl.BlockSpec((3, Wco, Wco), rep3),
            pl.BlockSpec((1, Wco), rep2),
        ],
        out_specs=(pl.BlockSpec((B, H, Wco), blk),
                   pl.BlockSpec((pl.Squeezed(), 2, Wco), blk)),
        scratch_shapes=[pltpu.VMEM((rows, Wci), bf16),
                        pltpu.VMEM((rows, Wco), bf16)],
        compiler_params=par,
    )(x2d, w1b, b1r, wb1b, bb1r)

    count = N * H * W
    s = stats.sum(axis=0).reshape(2, W, Cout).sum(axis=1)
    mean = s[0] / count
    var = s[1] / count - mean * mean
    scale = gamma / jnp.sqrt(var + _EPS)
    shift = beta - mean * scale
    scr, shr = _row(scale, W), _row(shift, W)

    out2d, down2d = pl.pallas_call(
        _phase2,
        out_shape=(jax.ShapeDtypeStruct((N, H, Wco), f32),
                   jax.ShapeDtypeStruct((N, H // 2, Wcd), f32)),
        grid=(G,),
        in_specs=[
            pl.BlockSpec((B, H, Wco), blk),
            pl.BlockSpec((B, H, Wci), blk),
            pl.BlockSpec((1, Wco), rep2),
            pl.BlockSpec((1, Wco), rep2),
            pl.BlockSpec((3, Wco, Wco), rep3),
            pl.BlockSpec((1, Wco), rep2),
            pl.BlockSpec((Wci, Wco), rep2),
            pl.BlockSpec((1, Wco), rep2),
            pl.BlockSpec((3, Wco, Wco), rep3),
            pl.BlockSpec((1, Wco), rep2),
            pl.BlockSpec((H // 2, H), rep2),
            pl.BlockSpec((Wco, Wcd), rep2),
            pl.BlockSpec((1, Wcd), rep2),
        ],
        out_specs=(pl.BlockSpec((B, H, Wco), blk),
                   pl.BlockSpec((B, H // 2, Wcd), blk)),
        scratch_shapes=[pltpu.VMEM((rows, Wco), bf16)],
        compiler_params=par,
    )(t2d, x2d, scr, shr, wb2b, bb2r, wsb, bsr, w2b, b2r, selb, wdb, bdr)

    out = out2d.reshape(N, H, W, Cout)
    down = down2d.reshape(N, H // 2, W // 2, Cout)
    return out, down


# speedup vs baseline: 1.0122x; 1.0122x over previous
import jax
import jax.numpy as jnp
from jax.experimental import pallas as pl
from jax.experimental.pallas import tpu as pltpu

_EPS = 1e-5
_IMGS = 32
_GRP = 8
_OFF = 16


def _fold3x3(w, W):
    shift = jnp.stack([jnp.eye(W, W, k=1 - dx, dtype=w.dtype) for dx in range(3)])
    band = jnp.einsum("dij,ydab->yiajb", shift, w)
    return band.reshape(3, W * w.shape[2], W * w.shape[3])


def _fold1x1(w, W):
    return jnp.kron(jnp.eye(W, dtype=w.dtype), w)


def _fold1x1_s2(w, W):
    pick = jnp.eye(W, dtype=w.dtype)[:, ::2]
    return jnp.einsum("ij,ab->iajb", pick, w).reshape(W * w.shape[0], (W // 2) * w.shape[1])


def _row(v, W):
    return jnp.tile(v.astype(jnp.float32), W)[None, :]


def _dot(a, b):
    return jnp.dot(a, b, preferred_element_type=jnp.float32)


def _scatter(pad_ref, imgs, H):
    seg = H + _OFF
    zero = jnp.zeros((1, pad_ref.shape[1]), pad_ref.dtype)
    for i, img in enumerate(imgs):
        base = i * seg
        pad_ref[base:base + 1, :] = zero
        pad_ref[base + _OFF - 1:base + _OFF, :] = zero
        pad_ref[base + _OFF:base + _OFF + H, :] = img
    pad_ref[len(imgs) * seg:len(imgs) * seg + 1, :] = zero


def _band_conv(pad_ref, w_ref, M):
    acc = _dot(pad_ref[_OFF - 1:_OFF - 1 + M, :], w_ref[0])
    acc += _dot(pad_ref[_OFF:_OFF + M, :], w_ref[1])
    acc += _dot(pad_ref[_OFF + 1:_OFF + 1 + M, :], w_ref[2])
    return acc


def _phase1(x_ref, w1_ref, b1_ref, wb1_ref, bb1_ref, t_ref, st_ref,
            pa_ref, pb_ref):
    B, H = x_ref.shape[0], x_ref.shape[1]
    seg = H + _OFF
    P = min(_GRP, B)
    M = (P - 1) * seg + H + 2

    s1 = jnp.zeros((1, t_ref.shape[2]), jnp.float32)
    s2 = jnp.zeros((1, t_ref.shape[2]), jnp.float32)
    for g in range(0, B, P):
        _scatter(pa_ref, [x_ref[g + i].astype(pa_ref.dtype) for i in range(P)], H)
        x1 = jnp.maximum(_band_conv(pa_ref, w1_ref, M) + b1_ref[...], 0.0)

        _scatter(pb_ref, [x1[i * seg:i * seg + H].astype(pb_ref.dtype)
                          for i in range(P)], H)
        t = _band_conv(pb_ref, wb1_ref, M) + bb1_ref[...]

        for i in range(P):
            ti = t[i * seg:i * seg + H]
            t_ref[g + i] = ti.astype(t_ref.dtype)
            s1 += jnp.sum(ti, axis=0, keepdims=True)
            s2 += jnp.sum(ti * ti, axis=0, keepdims=True)
    st_ref[0:1, :] = s1
    st_ref[1:2, :] = s2


def _phase2(t_ref, x_ref, sc_ref, sh_ref, wb2_ref, bb2_ref, ws_ref, bs_ref,
            w2_ref, b2_ref, sel_ref, wd_ref, bd_ref, out_ref, dn_ref, pa_ref):
    B, H = t_ref.shape[0], t_ref.shape[1]
    seg = H + _OFF
    P = min(_GRP, B)
    M = (P - 1) * seg + H + 2
    Hh = H // 2
    bf16 = jnp.bfloat16

    for g in range(0, B, P):
        tb = [jnp.maximum(t_ref[g + i].astype(jnp.float32) * sc_ref[...]
                          + sh_ref[...], 0.0).astype(bf16) for i in range(P)]
        _scatter(pa_ref, tb, H)
        acc = _band_conv(pa_ref, wb2_ref, M) + bb2_ref[...]

        xs = _dot(x_ref[g:g + P].reshape(P * H, x_ref.shape[2]).astype(bf16),
                  ws_ref[...]) + bs_ref[...]

        x3 = [(acc[i * seg:i * seg + H] + xs[i * H:(i + 1) * H]).astype(bf16)
              for i in range(P)]
        _scatter(pa_ref, x3, H)
        out = jnp.maximum(_band_conv(pa_ref, w2_ref, M) + b2_ref[...], 0.0)

        dec = []
        for i in range(P):
            oi = out[i * seg:i * seg + H]
            out_ref[g + i] = oi
            dec.append(_dot(sel_ref[...], oi.astype(bf16)).astype(bf16))
        dn = _dot(jnp.concatenate(dec, axis=0), wd_ref[...]) + bd_ref[...]
        for i in range(P):
            dn_ref[g + i] = dn[i * Hh:(i + 1) * Hh]


def kernel(x_nhwc, w1, b1, wb1, bb1, gamma, beta, wb2, bb2, ws, bs, w2, b2, wd, bd):
    N, H, W, Cin = x_nhwc.shape
    Cout = w1.shape[-1]
    f32, bf16 = jnp.float32, jnp.bfloat16
    Wci, Wco, Wcd = W * Cin, W * Cout, (W // 2) * Cout

    B = _IMGS
    while N % B:
        B //= 2
    G = N // B
    rows = min(_GRP, B) * (H + _OFF) + 16

    x2d = x_nhwc.reshape(N, H, Wci)

    w1b = _fold3x3(w1, W).astype(bf16)
    wb1b = _fold3x3(wb1, W).astype(bf16)
    wb2b = _fold3x3(wb2, W).astype(bf16)
    w2b = _fold3x3(w2, W).astype(bf16)
    wsb = _fold1x1(ws, W).astype(bf16)
    wdb = _fold1x1_s2(wd, W).astype(bf16)
    selb = jnp.eye(H, dtype=bf16)[::2, :]
    b1r, bb1r = _row(b1, W), _row(bb1, W)
    bb2r, bsr = _row(bb2, W), _row(bs, W)
    b2r, bdr = _row(b2, W), _row(bd, W // 2)

    par = pltpu.CompilerParams(dimension_semantics=("parallel",))
    rep2 = lambda g: (0, 0)
    rep3 = lambda g: (0, 0, 0)
    blk = lambda g: (g, 0, 0)

    t2d, stats = pl.pallas_call(
        _phase1,
        out_shape=(jax.ShapeDtypeStruct((N, H, Wco), bf16),
                   jax.ShapeDtypeStruct((G, 2, Wco), f32)),
        grid=(G,),
        in_specs=[
            pl.BlockSpec((B, H, Wci), blk),
            pl.BlockSpec((3, Wci, Wco), rep3),
            pl.BlockSpec((1, Wco), rep2),
            pl.BlockSpec((3, Wco, Wco), rep3),
            pl.BlockSpec((1, Wco), rep2),
        ],
        out_specs=(pl.BlockSpec((B, H, Wco), blk),
                   pl.BlockSpec((pl.Squeezed(), 2, Wco), blk)),
        scratch_shapes=[pltpu.VMEM((rows, Wci), bf16),
                        pltpu.VMEM((rows, Wco), bf16)],
        compiler_params=par,
    )(x2d, w1b, b1r, wb1b, bb1r)

    count = N * H * W
    s = stats.sum(axis=0).reshape(2, W, Cout).sum(axis=1)
    mean = s[0] / count
    var = s[1] / count - mean * mean
    scale = gamma / jnp.sqrt(var + _EPS)
    shift = beta - mean * scale
    scr, shr = _row(scale, W), _row(shift, W)

    out2d, down2d = pl.pallas_call(
        _phase2,
        out_shape=(jax.ShapeDtypeStruct((N, H, Wco), f32),
                   jax.ShapeDtypeStruct((N, H // 2, Wcd), f32)),
        grid=(G,),
        in_specs=[
            pl.BlockSpec((B, H, Wco), blk),
            pl.BlockSpec((B, H, Wci), blk),
            pl.BlockSpec((1, Wco), rep2),
            pl.BlockSpec((1, Wco), rep2),
            pl.BlockSpec((3, Wco, Wco), rep3),
            pl.BlockSpec((1, Wco), rep2),
            pl.BlockSpec((Wci, Wco), rep2),
            pl.BlockSpec((1, Wco), rep2),
            pl.BlockSpec((3, Wco, Wco), rep3),
            pl.BlockSpec((1, Wco), rep2),
            pl.BlockSpec((H // 2, H), rep2),
            pl.BlockSpec((Wco, Wcd), rep2),
            pl.BlockSpec((1, Wcd), rep2),
        ],
        out_specs=(pl.BlockSpec((B, H, Wco), blk),
                   pl.BlockSpec((B, H // 2, Wcd), blk)),
        scratch_shapes=[pltpu.VMEM((rows, Wco), bf16)],
        compiler_params=par,
    )(t2d, x2d, scr, shr, wb2b, bb2r, wsb, bsr, w2b, b2r, selb, wdb, bdr)

    out = out2d.reshape(N, H, W, Cout)
    down = down2d.reshape(N, H // 2, W // 2, Cout)
    return out, down


# speedup vs baseline: 1.0253x; 1.0130x over previous
import jax
import jax.numpy as jnp
from jax.experimental import pallas as pl
from jax.experimental.pallas import tpu as pltpu

_EPS = 1e-5
_IMGS = 16
_GRP = 8
_OFF = 16


def _fold3x3(w, W):
    shift = jnp.stack([jnp.eye(W, W, k=1 - dx, dtype=w.dtype) for dx in range(3)])
    band = jnp.einsum("dij,ydab->yiajb", shift, w)
    return band.reshape(3, W * w.shape[2], W * w.shape[3])


def _fold1x1(w, W):
    return jnp.kron(jnp.eye(W, dtype=w.dtype), w)


def _fold1x1_s2(w, W):
    pick = jnp.eye(W, dtype=w.dtype)[:, ::2]
    return jnp.einsum("ij,ab->iajb", pick, w).reshape(W * w.shape[0], (W // 2) * w.shape[1])


def _row(v, W):
    return jnp.tile(v.astype(jnp.float32), W)[None, :]


def _dot(a, b):
    return jnp.dot(a, b, preferred_element_type=jnp.float32)


def _scatter(pad_ref, imgs, H):
    seg = H + _OFF
    zero = jnp.zeros((1, pad_ref.shape[1]), pad_ref.dtype)
    for i, img in enumerate(imgs):
        base = i * seg
        pad_ref[base:base + 1, :] = zero
        pad_ref[base + _OFF - 1:base + _OFF, :] = zero
        pad_ref[base + _OFF:base + _OFF + H, :] = img
    pad_ref[len(imgs) * seg:len(imgs) * seg + 1, :] = zero


def _band_conv(pad_ref, w_ref, M):
    acc = _dot(pad_ref[_OFF - 1:_OFF - 1 + M, :], w_ref[0])
    acc += _dot(pad_ref[_OFF:_OFF + M, :], w_ref[1])
    acc += _dot(pad_ref[_OFF + 1:_OFF + 1 + M, :], w_ref[2])
    return acc


def _phase1(x_ref, w1_ref, b1_ref, wb1_ref, bb1_ref, t_ref, st_ref,
            pa_ref, pb_ref):
    B, H = x_ref.shape[0], x_ref.shape[1]
    seg = H + _OFF
    P = min(_GRP, B)
    M = (P - 1) * seg + H + 2

    s1 = jnp.zeros((1, t_ref.shape[2]), jnp.float32)
    s2 = jnp.zeros((1, t_ref.shape[2]), jnp.float32)
    for g in range(0, B, P):
        _scatter(pa_ref, [x_ref[g + i].astype(pa_ref.dtype) for i in range(P)], H)
        x1 = jnp.maximum(_band_conv(pa_ref, w1_ref, M) + b1_ref[...], 0.0)

        _scatter(pb_ref, [x1[i * seg:i * seg + H].astype(pb_ref.dtype)
                          for i in range(P)], H)
        t = _band_conv(pb_ref, wb1_ref, M) + bb1_ref[...]

        for i in range(P):
            ti = t[i * seg:i * seg + H]
            t_ref[g + i] = ti.astype(t_ref.dtype)
            s1 += jnp.sum(ti, axis=0, keepdims=True)
            s2 += jnp.sum(ti * ti, axis=0, keepdims=True)
    st_ref[0:1, :] = s1
    st_ref[1:2, :] = s2


def _phase2(t_ref, x_ref, sc_ref, sh_ref, wb2_ref, bb2_ref, ws_ref, bs_ref,
            w2_ref, b2_ref, sel_ref, wd_ref, bd_ref, out_ref, dn_ref, pa_ref):
    B, H = t_ref.shape[0], t_ref.shape[1]
    seg = H + _OFF
    P = min(_GRP, B)
    M = (P - 1) * seg + H + 2
    Hh = H // 2
    bf16 = jnp.bfloat16

    for g in range(0, B, P):
        tb = [jnp.maximum(t_ref[g + i].astype(jnp.float32) * sc_ref[...]
                          + sh_ref[...], 0.0).astype(bf16) for i in range(P)]
        _scatter(pa_ref, tb, H)
        acc = _band_conv(pa_ref, wb2_ref, M) + bb2_ref[...]

        xs = _dot(x_ref[g:g + P].reshape(P * H, x_ref.shape[2]).astype(bf16),
                  ws_ref[...]) + bs_ref[...]

        x3 = [(acc[i * seg:i * seg + H] + xs[i * H:(i + 1) * H]).astype(bf16)
              for i in range(P)]
        _scatter(pa_ref, x3, H)
        out = jnp.maximum(_band_conv(pa_ref, w2_ref, M) + b2_ref[...], 0.0)

        dec = []
        for i in range(P):
            oi = out[i * seg:i * seg + H]
            out_ref[g + i] = oi
            dec.append(_dot(sel_ref[...], oi.astype(bf16)).astype(bf16))
        dn = _dot(jnp.concatenate(dec, axis=0), wd_ref[...]) + bd_ref[...]
        for i in range(P):
            dn_ref[g + i] = dn[i * Hh:(i + 1) * Hh]


def kernel(x_nhwc, w1, b1, wb1, bb1, gamma, beta, wb2, bb2, ws, bs, w2, b2, wd, bd):
    N, H, W, Cin = x_nhwc.shape
    Cout = w1.shape[-1]
    f32, bf16 = jnp.float32, jnp.bfloat16
    Wci, Wco, Wcd = W * Cin, W * Cout, (W // 2) * Cout

    B = _IMGS
    while N % B:
        B //= 2
    G = N // B
    rows = min(_GRP, B) * (H + _OFF) + 16

    x2d = x_nhwc.reshape(N, H, Wci)

    w1b = _fold3x3(w1, W).astype(bf16)
    wb1b = _fold3x3(wb1, W).astype(bf16)
    wb2b = _fold3x3(wb2, W).astype(bf16)
    w2b = _fold3x3(w2, W).astype(bf16)
    wsb = _fold1x1(ws, W).astype(bf16)
    wdb = _fold1x1_s2(wd, W).astype(bf16)
    selb = jnp.eye(H, dtype=bf16)[::2, :]
    b1r, bb1r = _row(b1, W), _row(bb1, W)
    bb2r, bsr = _row(bb2, W), _row(bs, W)
    b2r, bdr = _row(b2, W), _row(bd, W // 2)

    par = pltpu.CompilerParams(dimension_semantics=("parallel",))
    rep2 = lambda g: (0, 0)
    rep3 = lambda g: (0, 0, 0)
    blk = lambda g: (g, 0, 0)

    t2d, stats = pl.pallas_call(
        _phase1,
        out_shape=(jax.ShapeDtypeStruct((N, H, Wco), bf16),
                   jax.ShapeDtypeStruct((G, 2, Wco), f32)),
        grid=(G,),
        in_specs=[
            pl.BlockSpec((B, H, Wci), blk),
            pl.BlockSpec((3, Wci, Wco), rep3),
            pl.BlockSpec((1, Wco), rep2),
            pl.BlockSpec((3, Wco, Wco), rep3),
            pl.BlockSpec((1, Wco), rep2),
        ],
        out_specs=(pl.BlockSpec((B, H, Wco), blk),
                   pl.BlockSpec((pl.Squeezed(), 2, Wco), blk)),
        scratch_shapes=[pltpu.VMEM((rows, Wci), bf16),
                        pltpu.VMEM((rows, Wco), bf16)],
        compiler_params=par,
    )(x2d, w1b, b1r, wb1b, bb1r)

    count = N * H * W
    s = stats.sum(axis=0).reshape(2, W, Cout).sum(axis=1)
    mean = s[0] / count
    var = s[1] / count - mean * mean
    scale = gamma / jnp.sqrt(var + _EPS)
    shift = beta - mean * scale
    scr, shr = _row(scale, W), _row(shift, W)

    out2d, down2d = pl.pallas_call(
        _phase2,
        out_shape=(jax.ShapeDtypeStruct((N, H, Wco), f32),
                   jax.ShapeDtypeStruct((N, H // 2, Wcd), f32)),
        grid=(G,),
        in_specs=[
            pl.BlockSpec((B, H, Wco), blk),
            pl.BlockSpec((B, H, Wci), blk),
            pl.BlockSpec((1, Wco), rep2),
            pl.BlockSpec((1, Wco), rep2),
            pl.BlockSpec((3, Wco, Wco), rep3),
            pl.BlockSpec((1, Wco), rep2),
            pl.BlockSpec((Wci, Wco), rep2),
            pl.BlockSpec((1, Wco), rep2),
            pl.BlockSpec((3, Wco, Wco), rep3),
            pl.BlockSpec((1, Wco), rep2),
            pl.BlockSpec((H // 2, H), rep2),
            pl.BlockSpec((Wco, Wcd), rep2),
            pl.BlockSpec((1, Wcd), rep2),
        ],
        out_specs=(pl.BlockSpec((B, H, Wco), blk),
                   pl.BlockSpec((B, H // 2, Wcd), blk)),
        scratch_shapes=[pltpu.VMEM((rows, Wco), bf16)],
        compiler_params=par,
    )(t2d, x2d, scr, shr, wb2b, bb2r, wsb, bsr, w2b, b2r, selb, wdb, bdr)

    out = out2d.reshape(N, H, W, Cout)
    down = down2d.reshape(N, H // 2, W // 2, Cout)
    return out, down
